# baseline (speedup 1.0000x reference)
import functools

import jax
import jax.numpy as jnp
from jax import lax
from jax.experimental import pallas as pl
from jax.experimental.pallas import tpu as pltpu
from jax.experimental.pallas import tpu_sc as plsc

N = 10000
E = 320000
D_IN = 128
H = 256
N_STEPS = 4

NC = 2
NS = 16
NW = NC * NS
CA = 80
CB = 80
NPAD = 10240
EA = E // NW
EB = E // NS

_f32 = jnp.float32
_i32 = jnp.int32

_mesh = plsc.VectorSubcoreMesh(core_axis_name="c", subcore_axis_name="s")
_sc_params = pltpu.CompilerParams(use_tc_tiling_on_sc=False,
                                  needs_layout_passes=False)


@functools.partial(
    pl.kernel,
    out_type=(
        jax.ShapeDtypeStruct((E,), _f32),
        jax.ShapeDtypeStruct((NC, NPAD), _f32),
    ),
    mesh=_mesh,
    scratch_types=[
        pltpu.VMEM((CA, H), _f32),
        pltpu.VMEM((CA, H), _f32),
        pltpu.VMEM((CA,), _i32),
        pltpu.VMEM((CA,), _i32),
        pltpu.VMEM((CA,), _f32),
        pltpu.VMEM((640,), _f32),
        pltpu.VMEM_SHARED((NPAD,), _f32),
        pltpu.SemaphoreType.DMA,
    ],
    compiler_params=_sc_params,
)
def _sc_scores(q_hbm, k_hbm, src_hbm, dst_hbm, e_out, den_out,
               qrows, krows, dstb, srcb, ebuf, zbuf, den_sh, sem):
    c = lax.axis_index("c")
    s = lax.axis_index("s")
    wid = c * NS + s

    def _z(i, _):
        zbuf[pl.ds(i * 16, 16)] = jnp.zeros((16,), _f32)
        return 0
    lax.fori_loop(0, 40, _z, 0)
    pltpu.sync_copy(zbuf, den_sh.at[pl.ds(s * 640, 640)])
    plsc.subcore_barrier()

    ebase = wid * EA
    iota = lax.iota(_i32, 16)

    def _chunk(ci, _):
        off = ebase + ci * CA
        pltpu.sync_copy(dst_hbm.at[pl.ds(off, CA)], dstb)
        pltpu.sync_copy(src_hbm.at[pl.ds(off, CA)], srcb)
        d1 = pltpu.async_copy(q_hbm.at[dstb], qrows, sem)
        d2 = pltpu.async_copy(k_hbm.at[srcb], krows, sem)
        d1.wait()
        d2.wait()

        def _grp(g, _):
            eids = g * 16 + iota
            acc = jnp.zeros((16,), _f32)
            for d in range(H):
                dd = jnp.full((16,), d, _i32)
                qv = plsc.load_gather(qrows, [eids, dd])
                kv = plsc.load_gather(krows, [eids, dd])
                acc = acc + qv * kv
            ebuf[pl.ds(g * 16, 16)] = jnp.exp(acc * 0.0625)
            return 0
        lax.fori_loop(0, CA // 16, _grp, 0)
        pltpu.sync_copy(ebuf, e_out.at[pl.ds(off, CA)])
        pltpu.sync_copy(ebuf, den_sh.at[dstb], add=True)
        return 0
    lax.fori_loop(0, EA // CA, _chunk, 0)

    plsc.subcore_barrier()
    pltpu.sync_copy(den_sh.at[pl.ds(s * 640, 640)],
                    den_out.at[c, pl.ds(s * 640, 640)])


@functools.partial(
    pl.kernel,
    out_type=jax.ShapeDtypeStruct((NC * N, 128), _f32),
    mesh=_mesh,
    scratch_types=[
        pltpu.VMEM((CB, 128), _f32),
        pltpu.VMEM((CB,), _i32),
        pltpu.VMEM((CB,), _i32),
        pltpu.VMEM((CB,), _f32),
        pltpu.VMEM((NPAD,), _f32),
        pltpu.VMEM((NPAD,), _f32),
        pltpu.VMEM((16, 128), _f32),
        pltpu.VMEM_SHARED((NPAD, 128), _f32),
        pltpu.SemaphoreType.DMA,
    ],
    compiler_params=_sc_params,
)
def _sc_agg(vh_hbm, src_hbm, dst_hbm, e_hbm, den_hbm, agg_out,
            vrows, dstb, idxb, ebuf, denv, dtmp, zrows, agg_sh, sem):
    c = lax.axis_index("c")
    s = lax.axis_index("s")

    pltpu.sync_copy(den_hbm.at[0], denv)
    pltpu.sync_copy(den_hbm.at[1], dtmp)

    def _dsum(i, _):
        sl = pl.ds(i * 16, 16)
        denv[sl] = denv[sl] + dtmp[sl] + 1e-30
        return 0
    lax.fori_loop(0, NPAD // 16, _dsum, 0)

    for i in range(16):
        for j in range(8):
            zrows[i, pl.ds(j * 16, 16)] = jnp.zeros((16,), _f32)

    def _zblk(i, _):
        pltpu.sync_copy(zrows, agg_sh.at[pl.ds(s * 640 + i * 16, 16), :])
        return 0
    lax.fori_loop(0, 40, _zblk, 0)
    plsc.subcore_barrier()

    ebase = s * EB
    cbase = c * N
    iota = lax.iota(_i32, 16)

    def _chunk(ci, _):
        off = ebase + ci * CB
        pltpu.sync_copy(dst_hbm.at[pl.ds(off, CB)], dstb)
        pltpu.sync_copy(src_hbm.at[pl.ds(off, CB)], idxb)
        pltpu.sync_copy(e_hbm.at[pl.ds(off, CB)], ebuf)

        def _fix(i, _):
            sl = pl.ds(i * 16, 16)
            idxb[sl] = idxb[sl] + cbase
            return 0
        lax.fori_loop(0, CB // 16, _fix, 0)
        pltpu.async_copy(vh_hbm.at[idxb], vrows, sem).wait()

        def _grp(g, _):
            sl = pl.ds(g * 16, 16)
            den16 = plsc.load_gather(denv, [dstb[sl]])
            alpha = ebuf[sl] / den16
            eids = g * 16 + iota
            for d in range(128):
                dd = jnp.full((16,), d, _i32)
                vv = plsc.load_gather(vrows, [eids, dd])
                plsc.store_scatter(vrows, [eids, dd], vv * alpha)
            return 0
        lax.fori_loop(0, CB // 16, _grp, 0)
        pltpu.sync_copy(vrows, agg_sh.at[dstb], add=True)
        return 0
    lax.fori_loop(0, EB // CB, _chunk, 0)

    plsc.subcore_barrier()
    pltpu.sync_copy(agg_sh.at[pl.ds(s * 625, 625), :],
                    agg_out.at[pl.ds(cbase + s * 625, 625), :])


_RB = 1000


def _first_body(x_ref, we_ref, be_ref, w_ref, b_ref,
                h_ref, q_ref, k_ref, vh_ref, s_ref):
    h = jnp.dot(x_ref[...], we_ref[...], preferred_element_type=_f32) + be_ref[...]
    h_ref[...] = h
    o = jnp.dot(h, w_ref[...], preferred_element_type=_f32) + b_ref[...]
    q_ref[...] = o[:, 0:256]
    k_ref[...] = o[:, 256:512]
    vh_ref[0] = o[:, 512:640]
    vh_ref[1] = o[:, 640:768]
    s_ref[...] = o[:, 768:1024]


def _step_body(y_ref, aa_ref, ab_ref, sp_ref, dt_ref, w_ref, b_ref,
               y_out, q_ref, k_ref, vh_ref, s_ref):
    f = jnp.concatenate([aa_ref[...], ab_ref[...]], axis=1) + sp_ref[...]
    y = y_ref[...] + dt_ref[0, 0] * f
    y_out[...] = y
    o = jnp.dot(y, w_ref[...], preferred_element_type=_f32) + b_ref[...]
    q_ref[...] = o[:, 0:256]
    k_ref[...] = o[:, 256:512]
    vh_ref[0] = o[:, 512:640]
    vh_ref[1] = o[:, 640:768]
    s_ref[...] = o[:, 768:1024]


def _final_body(y_ref, aa_ref, ab_ref, sp_ref, dt_ref, y_out):
    f = jnp.concatenate([aa_ref[...], ab_ref[...]], axis=1) + sp_ref[...]
    y_out[...] = y_ref[...] + dt_ref[0, 0] * f


def _qkvs_out():
    return (
        jax.ShapeDtypeStruct((N, H), _f32),
        jax.ShapeDtypeStruct((N, H), _f32),
        jax.ShapeDtypeStruct((2, N, 128), _f32),
        jax.ShapeDtypeStruct((N, H), _f32),
    )


def _qkvs_specs():
    return [
        pl.BlockSpec((_RB, H), lambda i: (i, 0)),
        pl.BlockSpec((_RB, H), lambda i: (i, 0)),
        pl.BlockSpec((2, _RB, 128), lambda i: (0, i, 0)),
        pl.BlockSpec((_RB, H), lambda i: (i, 0)),
    ]


def _tc_first(x, we, be, wcat, bcat):
    return pl.pallas_call(
        _first_body,
        grid=(N // _RB,),
        in_specs=[
            pl.BlockSpec((_RB, D_IN), lambda i: (i, 0)),
            pl.BlockSpec((D_IN, H), lambda i: (0, 0)),
            pl.BlockSpec((1, H), lambda i: (0, 0)),
            pl.BlockSpec((H, 4 * H), lambda i: (0, 0)),
            pl.BlockSpec((1, 4 * H), lambda i: (0, 0)),
        ],
        out_specs=[pl.BlockSpec((_RB, H), lambda i: (i, 0))] + _qkvs_specs(),
        out_shape=(jax.ShapeDtypeStruct((N, H), _f32),) + _qkvs_out(),
    )(x, we, be, wcat, bcat)


def _tc_step(y, aggA, aggB, sp, dtv, wcat, bcat):
    return pl.pallas_call(
        _step_body,
        grid=(N // _RB,),
        in_specs=[
            pl.BlockSpec((_RB, H), lambda i: (i, 0)),
            pl.BlockSpec((_RB, 128), lambda i: (i, 0)),
            pl.BlockSpec((_RB, 128), lambda i: (i, 0)),
            pl.BlockSpec((_RB, H), lambda i: (i, 0)),
            pl.BlockSpec((1, 1), lambda i: (0, 0)),
            pl.BlockSpec((H, 4 * H), lambda i: (0, 0)),
            pl.BlockSpec((1, 4 * H), lambda i: (0, 0)),
        ],
        out_specs=[pl.BlockSpec((_RB, H), lambda i: (i, 0))] + _qkvs_specs(),
        out_shape=(jax.ShapeDtypeStruct((N, H), _f32),) + _qkvs_out(),
    )(y, aggA, aggB, sp, dtv, wcat, bcat)


def _tc_final(y, aggA, aggB, sp, dtv):
    return pl.pallas_call(
        _final_body,
        grid=(N // _RB,),
        in_specs=[
            pl.BlockSpec((_RB, H), lambda i: (i, 0)),
            pl.BlockSpec((_RB, 128), lambda i: (i, 0)),
            pl.BlockSpec((_RB, 128), lambda i: (i, 0)),
            pl.BlockSpec((_RB, H), lambda i: (i, 0)),
            pl.BlockSpec((1, 1), lambda i: (0, 0)),
        ],
        out_specs=pl.BlockSpec((_RB, H), lambda i: (i, 0)),
        out_shape=jax.ShapeDtypeStruct((N, H), _f32),
    )(y, aggA, aggB, sp, dtv)


def kernel(x, edge_index, W_emb, b_emb, Wq, bq, Wk, bk, Wv, bv, Ws, bs):
    src = edge_index[0]
    dst = edge_index[1]
    wcat = jnp.concatenate([Wq[1:], Wk[1:], Wv[1:], Ws[1:]], axis=1)
    w0 = jnp.concatenate([Wq[0], Wk[0], Wv[0], Ws[0]])
    bcat = jnp.concatenate([bq, bk, bv, bs])
    ts = jnp.linspace(0.0, 1.0, N_STEPS)

    b0 = (bcat + ts[0] * w0)[None, :]
    h, q, k, vh, sp = _tc_first(x, W_emb, b_emb[None, :], wcat, b0)
    ys = [h]
    y = h
    for i in range(N_STEPS - 1):
        e, denp = _sc_scores(q, k, src, dst)
        agg = _sc_agg(vh.reshape(2 * N, 128), src, dst, e, denp)
        dtv = (ts[i + 1] - ts[i]).reshape(1, 1)
        if i < N_STEPS - 2:
            bi = (bcat + ts[i + 1] * w0)[None, :]
            y, q, k, vh, sp = _tc_step(y, agg[:N], agg[N:], sp, dtv, wcat, bi)
        else:
            y = _tc_final(y, agg[:N], agg[N:], sp, dtv)
        ys.append(y)
    return jnp.stack(ys, axis=0)


# speedup vs baseline: 4.9611x; 4.9611x over previous
import functools

import jax
import jax.numpy as jnp
from jax import lax
from jax.experimental import pallas as pl
from jax.experimental.pallas import tpu as pltpu
from jax.experimental.pallas import tpu_sc as plsc

N = 10000
E = 320000
D_IN = 128
H = 256
N_STEPS = 4

NC = 2
NS = 16
NW = NC * NS
CA = 80
CB = 80
NPAD = 10240
EA = E // NW
EB = E // NS

_f32 = jnp.float32
_i32 = jnp.int32

_mesh = plsc.VectorSubcoreMesh(core_axis_name="c", subcore_axis_name="s")
_sc_params = pltpu.CompilerParams(use_tc_tiling_on_sc=False,
                                  needs_layout_passes=False)


@functools.partial(
    pl.kernel,
    out_type=(
        jax.ShapeDtypeStruct((E,), _f32),
        jax.ShapeDtypeStruct((NC, NPAD), _f32),
    ),
    mesh=_mesh,
    scratch_types=[
        pltpu.VMEM((CA, H), _f32),
        pltpu.VMEM((CA, H), _f32),
        pltpu.VMEM((CA, H), _f32),
        pltpu.VMEM((CA, H), _f32),
        pltpu.VMEM((CA,), _i32),
        pltpu.VMEM((CA,), _i32),
        pltpu.VMEM((CA,), _i32),
        pltpu.VMEM((CA,), _i32),
        pltpu.VMEM((CA,), _f32),
        pltpu.VMEM((640,), _f32),
        pltpu.VMEM_SHARED((NPAD,), _f32),
        pltpu.SemaphoreType.DMA,
        pltpu.SemaphoreType.DMA,
    ],
    compiler_params=_sc_params,
)
def _sc_scores(q_hbm, k_hbm, src_hbm, dst_hbm, e_out, den_out,
               qrows0, krows0, qrows1, krows1, dstb0, srcb0, dstb1, srcb1,
               ebuf, zbuf, den_sh, sem0, sem1):
    c = lax.axis_index("c")
    s = lax.axis_index("s")
    wid = c * NS + s

    def _z(i, _):
        zbuf[pl.ds(i * 16, 16)] = jnp.zeros((16,), _f32)
        return 0
    lax.fori_loop(0, 40, _z, 0)
    pltpu.sync_copy(zbuf, den_sh.at[pl.ds(s * 640, 640)])
    plsc.subcore_barrier()

    ebase = wid * EA
    iota = lax.iota(_i32, 16)
    bufs = ((qrows0, krows0, dstb0, srcb0, sem0),
            (qrows1, krows1, dstb1, srcb1, sem1))

    def _fire(ci, qr, kr, db, sb, sem):
        off = ebase + ci * CA
        pltpu.sync_copy(dst_hbm.at[pl.ds(off, CA)], db)
        pltpu.sync_copy(src_hbm.at[pl.ds(off, CA)], sb)
        pltpu.async_copy(q_hbm.at[db], qr, sem)
        pltpu.async_copy(k_hbm.at[sb], kr, sem)

    def _wait(qr, kr, db, sb, sem):
        pltpu.make_async_copy(q_hbm.at[db], qr, sem).wait()
        pltpu.make_async_copy(k_hbm.at[sb], kr, sem).wait()

    def _compute(ci, qr, kr, db, sb, sem):
        off = ebase + ci * CA

        def _grp(g, _):
            svec = jnp.zeros((16,), _f32)
            for t in range(16):
                e = g * 16 + t
                acc = jnp.zeros((16,), _f32)
                for j in range(H // 16):
                    sl = pl.ds(j * 16, 16)
                    acc = acc + qr[e, sl] * kr[e, sl]
                svec = jnp.where(iota == t, jnp.sum(acc), svec)
            ebuf[pl.ds(g * 16, 16)] = jnp.exp(svec * 0.0625)
            return 0
        lax.fori_loop(0, CA // 16, _grp, 0)
        pltpu.sync_copy(ebuf, e_out.at[pl.ds(off, CA)])
        pltpu.sync_copy(ebuf, den_sh.at[db], add=True)

    _fire(0, *bufs[0])

    def _pair(p, _):
        c0 = 2 * p
        _fire(c0 + 1, *bufs[1])
        _wait(*bufs[0])
        _compute(c0, *bufs[0])
        _fire(c0 + 2, *bufs[0])
        _wait(*bufs[1])
        _compute(c0 + 1, *bufs[1])
        return 0
    lax.fori_loop(0, (EA // CA - 1) // 2, _pair, 0)
    _wait(*bufs[0])
    _compute(EA // CA - 1, *bufs[0])

    plsc.subcore_barrier()
    pltpu.sync_copy(den_sh.at[pl.ds(s * 640, 640)],
                    den_out.at[c, pl.ds(s * 640, 640)])


@functools.partial(
    pl.kernel,
    out_type=jax.ShapeDtypeStruct((NC * N, 128), _f32),
    mesh=_mesh,
    scratch_types=[
        pltpu.VMEM((CB, 128), _f32),
        pltpu.VMEM((CB, 128), _f32),
        pltpu.VMEM((CB,), _i32),
        pltpu.VMEM((CB,), _i32),
        pltpu.VMEM((CB,), _i32),
        pltpu.VMEM((CB,), _i32),
        pltpu.VMEM((CB,), _f32),
        pltpu.VMEM((CB,), _f32),
        pltpu.VMEM((CB,), _f32),
        pltpu.VMEM((NPAD,), _f32),
        pltpu.VMEM((NPAD,), _f32),
        pltpu.VMEM((16, 128), _f32),
        pltpu.VMEM_SHARED((NPAD, 128), _f32),
        pltpu.SemaphoreType.DMA,
        pltpu.SemaphoreType.DMA,
    ],
    compiler_params=_sc_params,
)
def _sc_agg(vh_hbm, src_hbm, dst_hbm, e_hbm, den_hbm, agg_out,
            vrows0, vrows1, dstb0, idxb0, dstb1, idxb1, ebuf0, ebuf1,
            abuf, denv, dtmp, zrows, agg_sh, sem0, sem1):
    c = lax.axis_index("c")
    s = lax.axis_index("s")

    pltpu.sync_copy(den_hbm.at[0], denv)
    pltpu.sync_copy(den_hbm.at[1], dtmp)

    def _dsum(i, _):
        sl = pl.ds(i * 16, 16)
        denv[sl] = denv[sl] + dtmp[sl] + 1e-30
        return 0
    lax.fori_loop(0, NPAD // 16, _dsum, 0)

    for i in range(16):
        for j in range(8):
            zrows[i, pl.ds(j * 16, 16)] = jnp.zeros((16,), _f32)

    def _zblk(i, _):
        pltpu.sync_copy(zrows, agg_sh.at[pl.ds(s * 640 + i * 16, 16), :])
        return 0
    lax.fori_loop(0, 40, _zblk, 0)
    plsc.subcore_barrier()

    ebase = s * EB
    cbase = c * N
    bufs = ((vrows0, dstb0, idxb0, ebuf0, sem0),
            (vrows1, dstb1, idxb1, ebuf1, sem1))

    def _fire(ci, vr, db, ib, eb, sem):
        off = ebase + ci * CB
        pltpu.sync_copy(dst_hbm.at[pl.ds(off, CB)], db)
        pltpu.sync_copy(src_hbm.at[pl.ds(off, CB)], ib)
        pltpu.sync_copy(e_hbm.at[pl.ds(off, CB)], eb)

        def _fix(i, _):
            sl = pl.ds(i * 16, 16)
            ib[sl] = ib[sl] + cbase
            return 0
        lax.fori_loop(0, CB // 16, _fix, 0)
        pltpu.async_copy(vh_hbm.at[ib], vr, sem)

    def _wait(vr, db, ib, eb, sem):
        pltpu.make_async_copy(vh_hbm.at[ib], vr, sem).wait()

    def _compute(vr, db, ib, eb, sem):
        def _grp(g, _):
            sl = pl.ds(g * 16, 16)
            den16 = plsc.load_gather(denv, [db[sl]])
            abuf[sl] = eb[sl] / den16
            return 0
        lax.fori_loop(0, CB // 16, _grp, 0)

        def _scale(e, _):
            a = plsc.load_gather(abuf, [jnp.full((16,), e, _i32)])
            for j in range(128 // 16):
                sl = pl.ds(j * 16, 16)
                vr[e, sl] = vr[e, sl] * a
            return 0
        lax.fori_loop(0, CB, _scale, 0)
        pltpu.sync_copy(vr, agg_sh.at[db], add=True)

    nc = EB // CB
    _fire(0, *bufs[0])

    def _pair(p, _):
        c0 = 2 * p
        _fire(c0 + 1, *bufs[1])
        _wait(*bufs[0])
        _compute(*bufs[0])
        _fire(c0 + 2, *bufs[0])
        _wait(*bufs[1])
        _compute(*bufs[1])
        return 0
    lax.fori_loop(0, (nc - 2) // 2, _pair, 0)
    _wait(*bufs[0])
    _compute(*bufs[0])
    _fire(nc - 1, *bufs[1])
    _wait(*bufs[1])
    _compute(*bufs[1])

    plsc.subcore_barrier()
    pltpu.sync_copy(agg_sh.at[pl.ds(s * 625, 625), :],
                    agg_out.at[pl.ds(cbase + s * 625, 625), :])


_RB = 1000


def _first_body(x_ref, we_ref, be_ref, w_ref, b_ref,
                h_ref, q_ref, k_ref, vh_ref, s_ref):
    h = jnp.dot(x_ref[...], we_ref[...], preferred_element_type=_f32) + be_ref[...]
    h_ref[...] = h
    o = jnp.dot(h, w_ref[...], preferred_element_type=_f32) + b_ref[...]
    q_ref[...] = o[:, 0:256]
    k_ref[...] = o[:, 256:512]
    vh_ref[0] = o[:, 512:640]
    vh_ref[1] = o[:, 640:768]
    s_ref[...] = o[:, 768:1024]


def _step_body(y_ref, aa_ref, ab_ref, sp_ref, dt_ref, w_ref, b_ref,
               y_out, q_ref, k_ref, vh_ref, s_ref):
    f = jnp.concatenate([aa_ref[...], ab_ref[...]], axis=1) + sp_ref[...]
    y = y_ref[...] + dt_ref[0, 0] * f
    y_out[...] = y
    o = jnp.dot(y, w_ref[...], preferred_element_type=_f32) + b_ref[...]
    q_ref[...] = o[:, 0:256]
    k_ref[...] = o[:, 256:512]
    vh_ref[0] = o[:, 512:640]
    vh_ref[1] = o[:, 640:768]
    s_ref[...] = o[:, 768:1024]


def _final_body(y_ref, aa_ref, ab_ref, sp_ref, dt_ref, y_out):
    f = jnp.concatenate([aa_ref[...], ab_ref[...]], axis=1) + sp_ref[...]
    y_out[...] = y_ref[...] + dt_ref[0, 0] * f


def _qkvs_out():
    return (
        jax.ShapeDtypeStruct((N, H), _f32),
        jax.ShapeDtypeStruct((N, H), _f32),
        jax.ShapeDtypeStruct((2, N, 128), _f32),
        jax.ShapeDtypeStruct((N, H), _f32),
    )


def _qkvs_specs():
    return [
        pl.BlockSpec((_RB, H), lambda i: (i, 0)),
        pl.BlockSpec((_RB, H), lambda i: (i, 0)),
        pl.BlockSpec((2, _RB, 128), lambda i: (0, i, 0)),
        pl.BlockSpec((_RB, H), lambda i: (i, 0)),
    ]


def _tc_first(x, we, be, wcat, bcat):
    return pl.pallas_call(
        _first_body,
        grid=(N // _RB,),
        in_specs=[
            pl.BlockSpec((_RB, D_IN), lambda i: (i, 0)),
            pl.BlockSpec((D_IN, H), lambda i: (0, 0)),
            pl.BlockSpec((1, H), lambda i: (0, 0)),
            pl.BlockSpec((H, 4 * H), lambda i: (0, 0)),
            pl.BlockSpec((1, 4 * H), lambda i: (0, 0)),
        ],
        out_specs=[pl.BlockSpec((_RB, H), lambda i: (i, 0))] + _qkvs_specs(),
        out_shape=(jax.ShapeDtypeStruct((N, H), _f32),) + _qkvs_out(),
    )(x, we, be, wcat, bcat)


def _tc_step(y, aggA, aggB, sp, dtv, wcat, bcat):
    return pl.pallas_call(
        _step_body,
        grid=(N // _RB,),
        in_specs=[
            pl.BlockSpec((_RB, H), lambda i: (i, 0)),
            pl.BlockSpec((_RB, 128), lambda i: (i, 0)),
            pl.BlockSpec((_RB, 128), lambda i: (i, 0)),
            pl.BlockSpec((_RB, H), lambda i: (i, 0)),
            pl.BlockSpec((1, 1), lambda i: (0, 0)),
            pl.BlockSpec((H, 4 * H), lambda i: (0, 0)),
            pl.BlockSpec((1, 4 * H), lambda i: (0, 0)),
        ],
        out_specs=[pl.BlockSpec((_RB, H), lambda i: (i, 0))] + _qkvs_specs(),
        out_shape=(jax.ShapeDtypeStruct((N, H), _f32),) + _qkvs_out(),
    )(y, aggA, aggB, sp, dtv, wcat, bcat)


def _tc_final(y, aggA, aggB, sp, dtv):
    return pl.pallas_call(
        _final_body,
        grid=(N // _RB,),
        in_specs=[
            pl.BlockSpec((_RB, H), lambda i: (i, 0)),
            pl.BlockSpec((_RB, 128), lambda i: (i, 0)),
            pl.BlockSpec((_RB, 128), lambda i: (i, 0)),
            pl.BlockSpec((_RB, H), lambda i: (i, 0)),
            pl.BlockSpec((1, 1), lambda i: (0, 0)),
        ],
        out_specs=pl.BlockSpec((_RB, H), lambda i: (i, 0)),
        out_shape=jax.ShapeDtypeStruct((N, H), _f32),
    )(y, aggA, aggB, sp, dtv)


def kernel(x, edge_index, W_emb, b_emb, Wq, bq, Wk, bk, Wv, bv, Ws, bs):
    src = edge_index[0]
    dst = edge_index[1]
    wcat = jnp.concatenate([Wq[1:], Wk[1:], Wv[1:], Ws[1:]], axis=1)
    w0 = jnp.concatenate([Wq[0], Wk[0], Wv[0], Ws[0]])
    bcat = jnp.concatenate([bq, bk, bv, bs])
    ts = jnp.linspace(0.0, 1.0, N_STEPS)

    b0 = (bcat + ts[0] * w0)[None, :]
    h, q, k, vh, sp = _tc_first(x, W_emb, b_emb[None, :], wcat, b0)
    ys = [h]
    y = h
    for i in range(N_STEPS - 1):
        e, denp = _sc_scores(q, k, src, dst)
        agg = _sc_agg(vh.reshape(2 * N, 128), src, dst, e, denp)
        dtv = (ts[i + 1] - ts[i]).reshape(1, 1)
        if i < N_STEPS - 2:
            bi = (bcat + ts[i + 1] * w0)[None, :]
            y, q, k, vh, sp = _tc_step(y, agg[:N], agg[N:], sp, dtv, wcat, bi)
        else:
            y = _tc_final(y, agg[:N], agg[N:], sp, dtv)
        ys.append(y)
    return jnp.stack(ys, axis=0)


# speedup vs baseline: 5.9201x; 1.1933x over previous
import functools

import jax
import jax.numpy as jnp
from jax import lax
from jax.experimental import pallas as pl
from jax.experimental.pallas import tpu as pltpu
from jax.experimental.pallas import tpu_sc as plsc

N = 10000
E = 320000
D_IN = 128
H = 256
N_STEPS = 4

NC = 2
NS = 16
NW = NC * NS
CA = 80
CB = 80
NPAD = 10240
EA = E // NW
EB = E // NS

_f32 = jnp.float32
_i32 = jnp.int32

_mesh = plsc.VectorSubcoreMesh(core_axis_name="c", subcore_axis_name="s")
_sc_params = pltpu.CompilerParams(use_tc_tiling_on_sc=False,
                                  needs_layout_passes=False)


@functools.partial(
    pl.kernel,
    out_type=(
        jax.ShapeDtypeStruct((E,), _f32),
        jax.ShapeDtypeStruct((NC, NPAD), _f32),
    ),
    mesh=_mesh,
    scratch_types=[
        pltpu.VMEM((CA, H), _f32),
        pltpu.VMEM((CA, H), _f32),
        pltpu.VMEM((CA, H), _f32),
        pltpu.VMEM((CA, H), _f32),
        pltpu.VMEM((CA,), _i32),
        pltpu.VMEM((CA,), _i32),
        pltpu.VMEM((CA,), _i32),
        pltpu.VMEM((CA,), _i32),
        pltpu.VMEM((EA,), _f32),
        pltpu.VMEM((256,), _f32),
        pltpu.VMEM((640,), _f32),
        pltpu.VMEM_SHARED((NPAD,), _f32),
        pltpu.SemaphoreType.DMA,
        pltpu.SemaphoreType.DMA,
        pltpu.SemaphoreType.DMA,
        pltpu.SemaphoreType.DMA,
    ],
    compiler_params=_sc_params,
)
def _sc_scores(q_hbm, k_hbm, src_hbm, dst_hbm, e_out, den_out,
               qrows0, krows0, qrows1, krows1, dstb0, srcb0, dstb1, srcb1,
               estore, tbuf, zbuf, den_sh, semg0, semg1, semd0, semd1):
    c = lax.axis_index("c")
    s = lax.axis_index("s")
    wid = c * NS + s

    def _z(i, _):
        zbuf[pl.ds(i * 16, 16)] = jnp.zeros((16,), _f32)
        return 0
    lax.fori_loop(0, 40, _z, 0)
    pltpu.sync_copy(zbuf, den_sh.at[pl.ds(s * 640, 640)])
    plsc.subcore_barrier()

    ebase = wid * EA
    iota = lax.iota(_i32, 16)
    colidx = iota * 16
    bufs = ((qrows0, krows0, dstb0, srcb0, semg0, semd0),
            (qrows1, krows1, dstb1, srcb1, semg1, semd1))

    def _fire(ci, qr, kr, db, sb, semg, semd):
        @pl.when(ci >= 2)
        def _():
            pltpu.make_async_copy(
                estore.at[pl.ds(0, CA)], den_sh.at[db], semd).wait()
        off = ebase + ci * CA
        pltpu.sync_copy(dst_hbm.at[pl.ds(off, CA)], db)
        pltpu.sync_copy(src_hbm.at[pl.ds(off, CA)], sb)
        pltpu.async_copy(q_hbm.at[db], qr, semg)
        pltpu.async_copy(k_hbm.at[sb], kr, semg)

    def _wait(qr, kr, db, sb, semg, semd):
        pltpu.make_async_copy(q_hbm.at[db], qr, semg).wait()
        pltpu.make_async_copy(k_hbm.at[sb], kr, semg).wait()

    def _compute(ci, qr, kr, db, sb, semg, semd):
        off0 = ci * CA

        def _grp(g, _):
            for t in range(16):
                e = g * 16 + t
                a0 = jnp.zeros((16,), _f32)
                a1 = jnp.zeros((16,), _f32)
                for j in range(0, H // 16, 2):
                    a0 = a0 + qr[e, pl.ds(j * 16, 16)] * kr[e, pl.ds(j * 16, 16)]
                    a1 = a1 + (qr[e, pl.ds((j + 1) * 16, 16)]
                               * kr[e, pl.ds((j + 1) * 16, 16)])
                plsc.store_scatter(tbuf, [colidx + t], a0 + a1)
            svec = tbuf[pl.ds(0, 16)]
            for r in range(1, 16):
                svec = svec + tbuf[pl.ds(r * 16, 16)]
            estore[pl.ds(off0 + g * 16, 16)] = jnp.exp(svec * 0.0625)
            return 0
        lax.fori_loop(0, CA // 16, _grp, 0)
        pltpu.async_copy(estore.at[pl.ds(off0, CA)], den_sh.at[db], semd,
                         add=True)

    nca = EA // CA
    _fire(0, *bufs[0])

    def _pair(p, _):
        c0 = 2 * p
        _fire(c0 + 1, *bufs[1])
        _wait(*bufs[0])
        _compute(c0, *bufs[0])
        _fire(c0 + 2, *bufs[0])
        _wait(*bufs[1])
        _compute(c0 + 1, *bufs[1])
        return 0
    lax.fori_loop(0, (nca - 1) // 2, _pair, 0)
    _wait(*bufs[0])
    _compute(nca - 1, *bufs[0])

    pltpu.make_async_copy(estore.at[pl.ds(0, CA)], den_sh.at[dstb0],
                          semd0).wait()
    pltpu.make_async_copy(estore.at[pl.ds(0, CA)], den_sh.at[dstb1],
                          semd1).wait()
    pltpu.sync_copy(estore, e_out.at[pl.ds(ebase, EA)])
    plsc.subcore_barrier()
    pltpu.sync_copy(den_sh.at[pl.ds(s * 640, 640)],
                    den_out.at[c, pl.ds(s * 640, 640)])


def _densum_body(p_ref, o_ref):
    o_ref[...] = jnp.broadcast_to(
        jnp.sum(p_ref[...], axis=0, keepdims=True) + 1e-30, (8, NPAD))


def _tc_densum(denp):
    return pl.pallas_call(
        _densum_body,
        out_shape=jax.ShapeDtypeStruct((8, NPAD), _f32),
    )(denp)


@functools.partial(
    pl.kernel,
    out_type=jax.ShapeDtypeStruct((NC * N, 128), _f32),
    mesh=_mesh,
    scratch_types=[
        pltpu.VMEM((CB, 128), _f32),
        pltpu.VMEM((CB, 128), _f32),
        pltpu.VMEM((CB,), _i32),
        pltpu.VMEM((CB,), _i32),
        pltpu.VMEM((CB,), _i32),
        pltpu.VMEM((CB,), _i32),
        pltpu.VMEM((CB,), _f32),
        pltpu.VMEM((CB,), _f32),
        pltpu.VMEM((CB,), _f32),
        pltpu.VMEM((NPAD,), _f32),
        pltpu.VMEM((16, 128), _f32),
        pltpu.VMEM_SHARED((NPAD, 128), _f32),
        pltpu.SemaphoreType.DMA,
        pltpu.SemaphoreType.DMA,
        pltpu.SemaphoreType.DMA,
        pltpu.SemaphoreType.DMA,
    ],
    compiler_params=_sc_params,
)
def _sc_agg(vh_hbm, src_hbm, dst_hbm, e_hbm, den_hbm, agg_out,
            vrows0, vrows1, dstb0, idxb0, dstb1, idxb1, ebuf0, ebuf1,
            abuf, denv, zrows, agg_sh, semg0, semg1, sems0, sems1):
    c = lax.axis_index("c")
    s = lax.axis_index("s")

    pltpu.sync_copy(den_hbm.at[0], denv)

    for i in range(16):
        for j in range(8):
            zrows[i, pl.ds(j * 16, 16)] = jnp.zeros((16,), _f32)

    def _zblk(i, _):
        pltpu.sync_copy(zrows, agg_sh.at[pl.ds(s * 640 + i * 16, 16), :])
        return 0
    lax.fori_loop(0, 40, _zblk, 0)
    plsc.subcore_barrier()

    ebase = s * EB
    cbase = c * N
    bufs = ((vrows0, dstb0, idxb0, ebuf0, semg0, sems0),
            (vrows1, dstb1, idxb1, ebuf1, semg1, sems1))

    def _fire(ci, vr, db, ib, eb, semg, sems):
        @pl.when(ci >= 2)
        def _():
            pltpu.make_async_copy(vr, agg_sh.at[db], sems).wait()
        off = ebase + ci * CB
        pltpu.sync_copy(dst_hbm.at[pl.ds(off, CB)], db)
        pltpu.sync_copy(src_hbm.at[pl.ds(off, CB)], ib)
        pltpu.sync_copy(e_hbm.at[pl.ds(off, CB)], eb)

        def _fix(i, _):
            sl = pl.ds(i * 16, 16)
            ib[sl] = ib[sl] + cbase
            return 0
        lax.fori_loop(0, CB // 16, _fix, 0)
        pltpu.async_copy(vh_hbm.at[ib], vr, semg)

    def _wait(vr, db, ib, eb, semg, sems):
        pltpu.make_async_copy(vh_hbm.at[ib], vr, semg).wait()

    def _compute(vr, db, ib, eb, semg, sems):
        def _grp(g, _):
            sl = pl.ds(g * 16, 16)
            den16 = plsc.load_gather(denv, [db[sl]])
            abuf[sl] = eb[sl] / den16
            return 0
        lax.fori_loop(0, CB // 16, _grp, 0)

        def _scaleg(g, _):
            for t in range(16):
                e = g * 16 + t
                a = plsc.load_gather(abuf, [jnp.full((16,), 0, _i32) + e])
                for j in range(128 // 16):
                    sl = pl.ds(j * 16, 16)
                    vr[e, sl] = vr[e, sl] * a
            return 0
        lax.fori_loop(0, CB // 16, _scaleg, 0)
        pltpu.async_copy(vr, agg_sh.at[db], sems, add=True)

    ncb = EB // CB
    _fire(0, *bufs[0])

    def _pair(p, _):
        c0 = 2 * p
        _fire(c0 + 1, *bufs[1])
        _wait(*bufs[0])
        _compute(*bufs[0])
        _fire(c0 + 2, *bufs[0])
        _wait(*bufs[1])
        _compute(*bufs[1])
        return 0
    lax.fori_loop(0, (ncb - 2) // 2, _pair, 0)
    _wait(*bufs[0])
    _compute(*bufs[0])
    _fire(ncb - 1, *bufs[1])
    _wait(*bufs[1])
    _compute(*bufs[1])

    pltpu.make_async_copy(vrows0, agg_sh.at[dstb0], sems0).wait()
    pltpu.make_async_copy(vrows1, agg_sh.at[dstb1], sems1).wait()
    plsc.subcore_barrier()
    pltpu.sync_copy(agg_sh.at[pl.ds(s * 625, 625), :],
                    agg_out.at[pl.ds(cbase + s * 625, 625), :])


_RB = 1000


def _first_body(x_ref, we_ref, be_ref, w_ref, b_ref,
                h_ref, q_ref, k_ref, vh_ref, s_ref):
    h = jnp.dot(x_ref[...], we_ref[...], preferred_element_type=_f32) + be_ref[...]
    h_ref[...] = h
    o = jnp.dot(h, w_ref[...], preferred_element_type=_f32) + b_ref[...]
    q_ref[...] = o[:, 0:256]
    k_ref[...] = o[:, 256:512]
    vh_ref[0] = o[:, 512:640]
    vh_ref[1] = o[:, 640:768]
    s_ref[...] = o[:, 768:1024]


def _step_body(y_ref, aa_ref, ab_ref, sp_ref, dt_ref, w_ref, b_ref,
               y_out, q_ref, k_ref, vh_ref, s_ref):
    f = jnp.concatenate([aa_ref[...], ab_ref[...]], axis=1) + sp_ref[...]
    y = y_ref[...] + dt_ref[0, 0] * f
    y_out[...] = y
    o = jnp.dot(y, w_ref[...], preferred_element_type=_f32) + b_ref[...]
    q_ref[...] = o[:, 0:256]
    k_ref[...] = o[:, 256:512]
    vh_ref[0] = o[:, 512:640]
    vh_ref[1] = o[:, 640:768]
    s_ref[...] = o[:, 768:1024]


def _final_body(y_ref, aa_ref, ab_ref, sp_ref, dt_ref, y_out):
    f = jnp.concatenate([aa_ref[...], ab_ref[...]], axis=1) + sp_ref[...]
    y_out[...] = y_ref[...] + dt_ref[0, 0] * f


def _qkvs_out():
    return (
        jax.ShapeDtypeStruct((N, H), _f32),
        jax.ShapeDtypeStruct((N, H), _f32),
        jax.ShapeDtypeStruct((2, N, 128), _f32),
        jax.ShapeDtypeStruct((N, H), _f32),
    )


def _qkvs_specs():
    return [
        pl.BlockSpec((_RB, H), lambda i: (i, 0)),
        pl.BlockSpec((_RB, H), lambda i: (i, 0)),
        pl.BlockSpec((2, _RB, 128), lambda i: (0, i, 0)),
        pl.BlockSpec((_RB, H), lambda i: (i, 0)),
    ]


def _tc_first(x, we, be, wcat, bcat):
    return pl.pallas_call(
        _first_body,
        grid=(N // _RB,),
        in_specs=[
            pl.BlockSpec((_RB, D_IN), lambda i: (i, 0)),
            pl.BlockSpec((D_IN, H), lambda i: (0, 0)),
            pl.BlockSpec((1, H), lambda i: (0, 0)),
            pl.BlockSpec((H, 4 * H), lambda i: (0, 0)),
            pl.BlockSpec((1, 4 * H), lambda i: (0, 0)),
        ],
        out_specs=[pl.BlockSpec((_RB, H), lambda i: (i, 0))] + _qkvs_specs(),
        out_shape=(jax.ShapeDtypeStruct((N, H), _f32),) + _qkvs_out(),
    )(x, we, be, wcat, bcat)


def _tc_step(y, aggA, aggB, sp, dtv, wcat, bcat):
    return pl.pallas_call(
        _step_body,
        grid=(N // _RB,),
        in_specs=[
            pl.BlockSpec((_RB, H), lambda i: (i, 0)),
            pl.BlockSpec((_RB, 128), lambda i: (i, 0)),
            pl.BlockSpec((_RB, 128), lambda i: (i, 0)),
            pl.BlockSpec((_RB, H), lambda i: (i, 0)),
            pl.BlockSpec((1, 1), lambda i: (0, 0)),
            pl.BlockSpec((H, 4 * H), lambda i: (0, 0)),
            pl.BlockSpec((1, 4 * H), lambda i: (0, 0)),
        ],
        out_specs=[pl.BlockSpec((_RB, H), lambda i: (i, 0))] + _qkvs_specs(),
        out_shape=(jax.ShapeDtypeStruct((N, H), _f32),) + _qkvs_out(),
    )(y, aggA, aggB, sp, dtv, wcat, bcat)


def _tc_final(y, aggA, aggB, sp, dtv):
    return pl.pallas_call(
        _final_body,
        grid=(N // _RB,),
        in_specs=[
            pl.BlockSpec((_RB, H), lambda i: (i, 0)),
            pl.BlockSpec((_RB, 128), lambda i: (i, 0)),
            pl.BlockSpec((_RB, 128), lambda i: (i, 0)),
            pl.BlockSpec((_RB, H), lambda i: (i, 0)),
            pl.BlockSpec((1, 1), lambda i: (0, 0)),
        ],
        out_specs=pl.BlockSpec((_RB, H), lambda i: (i, 0)),
        out_shape=jax.ShapeDtypeStruct((N, H), _f32),
    )(y, aggA, aggB, sp, dtv)


def kernel(x, edge_index, W_emb, b_emb, Wq, bq, Wk, bk, Wv, bv, Ws, bs):
    src = edge_index[0]
    dst = edge_index[1]
    wcat = jnp.concatenate([Wq[1:], Wk[1:], Wv[1:], Ws[1:]], axis=1)
    w0 = jnp.concatenate([Wq[0], Wk[0], Wv[0], Ws[0]])
    bcat = jnp.concatenate([bq, bk, bv, bs])
    ts = jnp.linspace(0.0, 1.0, N_STEPS)

    b0 = (bcat + ts[0] * w0)[None, :]
    h, q, k, vh, sp = _tc_first(x, W_emb, b_emb[None, :], wcat, b0)
    ys = [h]
    y = h
    for i in range(N_STEPS - 1):
        e, denp = _sc_scores(q, k, src, dst)
        den2 = _tc_densum(denp)
        agg = _sc_agg(vh.reshape(2 * N, 128), src, dst, e, den2)
        dtv = (ts[i + 1] - ts[i]).reshape(1, 1)
        if i < N_STEPS - 2:
            bi = (bcat + ts[i + 1] * w0)[None, :]
            y, q, k, vh, sp = _tc_step(y, agg[:N], agg[N:], sp, dtv, wcat, bi)
        else:
            y = _tc_final(y, agg[:N], agg[N:], sp, dtv)
        ys.append(y)
    return jnp.stack(ys, axis=0)


# speedup vs baseline: 8.9151x; 1.5059x over previous
import functools

import jax
import jax.numpy as jnp
from jax import lax
from jax.experimental import pallas as pl
from jax.experimental.pallas import tpu as pltpu
from jax.experimental.pallas import tpu_sc as plsc

N = 10000
E = 320000
D_IN = 128
H = 256
N_STEPS = 4

NC = 2
NS = 16
NW = NC * NS
CA = 80
CB = 80
NPAD = 10240
EA = E // NW
EB = E // NS

_f32 = jnp.float32
_i32 = jnp.int32

_mesh = plsc.VectorSubcoreMesh(core_axis_name="c", subcore_axis_name="s")
_sc_params = pltpu.CompilerParams(use_tc_tiling_on_sc=False,
                                  needs_layout_passes=False)


@functools.partial(
    pl.kernel,
    out_type=(
        jax.ShapeDtypeStruct((E,), _f32),
        jax.ShapeDtypeStruct((NC, NPAD), _f32),
    ),
    mesh=_mesh,
    scratch_types=[
        pltpu.VMEM((3, CA, H), _f32),
        pltpu.VMEM((3, CA, H), _f32),
        pltpu.VMEM((3, CA), _i32),
        pltpu.VMEM((3, CA), _i32),
        pltpu.VMEM((3, CA), _f32),
        pltpu.VMEM((256,), _f32),
        pltpu.VMEM((640,), _f32),
        pltpu.VMEM_SHARED((NPAD,), _f32),
        pltpu.SemaphoreType.DMA,
        pltpu.SemaphoreType.DMA,
        pltpu.SemaphoreType.DMA,
        pltpu.SemaphoreType.DMA,
        pltpu.SemaphoreType.DMA,
        pltpu.SemaphoreType.DMA,
        pltpu.SemaphoreType.DMA,
        pltpu.SemaphoreType.DMA,
        pltpu.SemaphoreType.DMA,
        pltpu.SemaphoreType.DMA,
        pltpu.SemaphoreType.DMA,
        pltpu.SemaphoreType.DMA,
    ],
    compiler_params=_sc_params,
)
def _sc_scores(q_hbm, k_hbm, src_hbm, dst_hbm, e_out, den_out,
               qrows, krows, dstb, srcb, ebuf, tbuf, zbuf, den_sh,
               semi0, semi1, semi2, semg0, semg1, semg2,
               semd0, semd1, semd2, semo0, semo1, semo2):
    c = lax.axis_index("c")
    s = lax.axis_index("s")
    wid = c * NS + s

    def _z(i, _):
        zbuf[pl.ds(i * 16, 16)] = jnp.zeros((16,), _f32)
        return 0
    lax.fori_loop(0, 40, _z, 0)
    pltpu.sync_copy(zbuf, den_sh.at[pl.ds(s * 640, 640)])
    plsc.subcore_barrier()

    ebase = wid * EA
    iota = lax.iota(_i32, 16)
    colidx = iota * 16
    semi = (semi0, semi1, semi2)
    semg = (semg0, semg1, semg2)
    semd = (semd0, semd1, semd2)
    semo = (semo0, semo1, semo2)
    nca = EA // CA

    def _drain_scatter(j, b):
        pltpu.make_async_copy(
            ebuf.at[b], den_sh.at[dstb.at[b]], semd[b]).wait()
        pltpu.make_async_copy(
            ebuf.at[b], e_out.at[pl.ds(ebase, CA)], semo[b]).wait()

    def _fire_idx(ci, b):
        @pl.when(ci >= 3)
        def _():
            _drain_scatter(ci - 3, b)
        off = ebase + ci * CA
        pltpu.async_copy(dst_hbm.at[pl.ds(off, CA)], dstb.at[b], semi[b])
        pltpu.async_copy(src_hbm.at[pl.ds(off, CA)], srcb.at[b], semi[b])

    def _fire_gather(ci, b):
        pltpu.make_async_copy(
            dst_hbm.at[pl.ds(ebase, CA)], dstb.at[b], semi[b]).wait()
        pltpu.make_async_copy(
            src_hbm.at[pl.ds(ebase, CA)], srcb.at[b], semi[b]).wait()
        pltpu.async_copy(q_hbm.at[dstb.at[b]], qrows.at[b], semg[b])
        pltpu.async_copy(k_hbm.at[srcb.at[b]], krows.at[b], semg[b])

    def _compute(ci, b):
        pltpu.make_async_copy(
            q_hbm.at[dstb.at[b]], qrows.at[b], semg[b]).wait()
        pltpu.make_async_copy(
            k_hbm.at[srcb.at[b]], krows.at[b], semg[b]).wait()
        qr = qrows.at[b]
        kr = krows.at[b]
        eb = ebuf.at[b]

        def _grp(g, _):
            for t in range(16):
                e = g * 16 + t
                a0 = jnp.zeros((16,), _f32)
                a1 = jnp.zeros((16,), _f32)
                for j in range(0, H // 16, 2):
                    a0 = a0 + qr[e, pl.ds(j * 16, 16)] * kr[e, pl.ds(j * 16, 16)]
                    a1 = a1 + (qr[e, pl.ds((j + 1) * 16, 16)]
                               * kr[e, pl.ds((j + 1) * 16, 16)])
                plsc.store_scatter(tbuf, [colidx + t], a0 + a1)
            svec = tbuf[pl.ds(0, 16)]
            for r in range(1, 16):
                svec = svec + tbuf[pl.ds(r * 16, 16)]
            eb[pl.ds(g * 16, 16)] = jnp.exp(svec * 0.0625)
            return 0
        lax.fori_loop(0, CA // 16, _grp, 0)
        pltpu.async_copy(eb, den_sh.at[dstb.at[b]], semd[b], add=True)
        pltpu.async_copy(eb, e_out.at[pl.ds(ebase + ci * CA, CA)], semo[b])

    _fire_idx(0, 0)
    _fire_idx(1, 1)
    _fire_gather(0, 0)

    def _iter(m, _):
        for u in range(3):
            ci = 3 * m + u
            b = u

            @pl.when(ci + 2 < nca)
            def _():
                _fire_idx(ci + 2, (u + 2) % 3)

            @pl.when(ci + 1 < nca)
            def _():
                _fire_gather(ci + 1, (u + 1) % 3)

            @pl.when(ci < nca)
            def _():
                _compute(ci, b)
        return 0
    lax.fori_loop(0, (nca + 2) // 3, _iter, 0)

    _drain_scatter(nca - 3, (nca - 3) % 3)
    _drain_scatter(nca - 2, (nca - 2) % 3)
    _drain_scatter(nca - 1, (nca - 1) % 3)
    plsc.subcore_barrier()
    pltpu.sync_copy(den_sh.at[pl.ds(s * 640, 640)],
                    den_out.at[c, pl.ds(s * 640, 640)])


def _densum_body(p_ref, o_ref):
    o_ref[...] = jnp.broadcast_to(
        jnp.sum(p_ref[...], axis=0, keepdims=True) + 1e-30, (8, NPAD))


def _tc_densum(denp):
    return pl.pallas_call(
        _densum_body,
        out_shape=jax.ShapeDtypeStruct((8, NPAD), _f32),
    )(denp)


@functools.partial(
    pl.kernel,
    out_type=jax.ShapeDtypeStruct((NC * N, 128), _f32),
    mesh=_mesh,
    scratch_types=[
        pltpu.VMEM((3, CB, 128), _f32),
        pltpu.VMEM((3, CB), _i32),
        pltpu.VMEM((3, CB), _i32),
        pltpu.VMEM((3, CB), _f32),
        pltpu.VMEM((CB,), _f32),
        pltpu.VMEM((NPAD,), _f32),
        pltpu.VMEM((16, 128), _f32),
        pltpu.VMEM_SHARED((NPAD, 128), _f32),
        pltpu.SemaphoreType.DMA,
        pltpu.SemaphoreType.DMA,
        pltpu.SemaphoreType.DMA,
        pltpu.SemaphoreType.DMA,
        pltpu.SemaphoreType.DMA,
        pltpu.SemaphoreType.DMA,
        pltpu.SemaphoreType.DMA,
        pltpu.SemaphoreType.DMA,
        pltpu.SemaphoreType.DMA,
    ],
    compiler_params=_sc_params,
)
def _sc_agg(vh_hbm, src_hbm, dst_hbm, e_hbm, den_hbm, agg_out,
            vrows, dstb, idxb, ebuf, abuf, denv, zrows, agg_sh,
            semi0, semi1, semi2, semg0, semg1, semg2,
            sems0, sems1, sems2):
    c = lax.axis_index("c")
    s = lax.axis_index("s")

    pltpu.sync_copy(den_hbm.at[0], denv)

    for i in range(16):
        for j in range(8):
            zrows[i, pl.ds(j * 16, 16)] = jnp.zeros((16,), _f32)

    def _zblk(i, _):
        pltpu.sync_copy(zrows, agg_sh.at[pl.ds(s * 640 + i * 16, 16), :])
        return 0
    lax.fori_loop(0, 40, _zblk, 0)
    plsc.subcore_barrier()

    ebase = s * EB
    cbase = c * N
    semi = (semi0, semi1, semi2)
    semg = (semg0, semg1, semg2)
    sems = (sems0, sems1, sems2)
    ncb = EB // CB

    def _drain_scatter(b):
        pltpu.make_async_copy(
            vrows.at[b], agg_sh.at[dstb.at[b]], sems[b]).wait()

    def _fire_idx(ci, b):
        @pl.when(ci >= 3)
        def _():
            _drain_scatter(b)
        off = ebase + ci * CB
        pltpu.async_copy(dst_hbm.at[pl.ds(off, CB)], dstb.at[b], semi[b])
        pltpu.async_copy(src_hbm.at[pl.ds(off, CB)], idxb.at[b], semi[b])
        pltpu.async_copy(e_hbm.at[pl.ds(off, CB)], ebuf.at[b], semi[b])

    def _fire_gather(ci, b):
        for _ in range(3):
            pltpu.make_async_copy(
                e_hbm.at[pl.ds(ebase, CB)], ebuf.at[b], semi[b]).wait()
        ib = idxb.at[b]

        def _fix(i, _):
            sl = pl.ds(i * 16, 16)
            ib[sl] = ib[sl] + cbase
            return 0
        lax.fori_loop(0, CB // 16, _fix, 0)
        pltpu.async_copy(vh_hbm.at[ib], vrows.at[b], semg[b])

    def _compute(b):
        pltpu.make_async_copy(
            vh_hbm.at[idxb.at[b]], vrows.at[b], semg[b]).wait()
        vr = vrows.at[b]
        db = dstb.at[b]
        eb = ebuf.at[b]

        def _grp(g, _):
            sl = pl.ds(g * 16, 16)
            den16 = plsc.load_gather(denv, [db[sl]])
            abuf[sl] = eb[sl] / den16
            return 0
        lax.fori_loop(0, CB // 16, _grp, 0)

        def _scaleg(g, _):
            for t in range(16):
                e = g * 16 + t
                a = plsc.load_gather(abuf, [jnp.full((16,), 0, _i32) + e])
                for j in range(128 // 16):
                    sl = pl.ds(j * 16, 16)
                    vr[e, sl] = vr[e, sl] * a
            return 0
        lax.fori_loop(0, CB // 16, _scaleg, 0)
        pltpu.async_copy(vr, agg_sh.at[db], sems[b], add=True)

    _fire_idx(0, 0)
    _fire_idx(1, 1)
    _fire_gather(0, 0)

    def _iter(m, _):
        for u in range(3):
            ci = 3 * m + u
            b = u

            @pl.when(ci + 2 < ncb)
            def _():
                _fire_idx(ci + 2, (u + 2) % 3)

            @pl.when(ci + 1 < ncb)
            def _():
                _fire_gather(ci + 1, (u + 1) % 3)

            @pl.when(ci < ncb)
            def _():
                _compute(b)
        return 0
    lax.fori_loop(0, (ncb + 2) // 3, _iter, 0)

    _drain_scatter((ncb - 3) % 3)
    _drain_scatter((ncb - 2) % 3)
    _drain_scatter((ncb - 1) % 3)
    plsc.subcore_barrier()
    pltpu.sync_copy(agg_sh.at[pl.ds(s * 625, 625), :],
                    agg_out.at[pl.ds(cbase + s * 625, 625), :])


_RB = 1000


def _first_body(x_ref, we_ref, be_ref, w_ref, b_ref,
                h_ref, q_ref, k_ref, vh_ref, s_ref):
    h = jnp.dot(x_ref[...], we_ref[...], preferred_element_type=_f32) + be_ref[...]
    h_ref[...] = h
    o = jnp.dot(h, w_ref[...], preferred_element_type=_f32) + b_ref[...]
    q_ref[...] = o[:, 0:256]
    k_ref[...] = o[:, 256:512]
    vh_ref[0] = o[:, 512:640]
    vh_ref[1] = o[:, 640:768]
    s_ref[...] = o[:, 768:1024]


def _step_body(y_ref, aa_ref, ab_ref, sp_ref, dt_ref, w_ref, b_ref,
               y_out, q_ref, k_ref, vh_ref, s_ref):
    f = jnp.concatenate([aa_ref[...], ab_ref[...]], axis=1) + sp_ref[...]
    y = y_ref[...] + dt_ref[0, 0] * f
    y_out[...] = y
    o = jnp.dot(y, w_ref[...], preferred_element_type=_f32) + b_ref[...]
    q_ref[...] = o[:, 0:256]
    k_ref[...] = o[:, 256:512]
    vh_ref[0] = o[:, 512:640]
    vh_ref[1] = o[:, 640:768]
    s_ref[...] = o[:, 768:1024]


def _final_body(y_ref, aa_ref, ab_ref, sp_ref, dt_ref, y_out):
    f = jnp.concatenate([aa_ref[...], ab_ref[...]], axis=1) + sp_ref[...]
    y_out[...] = y_ref[...] + dt_ref[0, 0] * f


def _qkvs_out():
    return (
        jax.ShapeDtypeStruct((N, H), _f32),
        jax.ShapeDtypeStruct((N, H), _f32),
        jax.ShapeDtypeStruct((2, N, 128), _f32),
        jax.ShapeDtypeStruct((N, H), _f32),
    )


def _qkvs_specs():
    return [
        pl.BlockSpec((_RB, H), lambda i: (i, 0)),
        pl.BlockSpec((_RB, H), lambda i: (i, 0)),
        pl.BlockSpec((2, _RB, 128), lambda i: (0, i, 0)),
        pl.BlockSpec((_RB, H), lambda i: (i, 0)),
    ]


def _tc_first(x, we, be, wcat, bcat):
    return pl.pallas_call(
        _first_body,
        grid=(N // _RB,),
        in_specs=[
            pl.BlockSpec((_RB, D_IN), lambda i: (i, 0)),
            pl.BlockSpec((D_IN, H), lambda i: (0, 0)),
            pl.BlockSpec((1, H), lambda i: (0, 0)),
            pl.BlockSpec((H, 4 * H), lambda i: (0, 0)),
            pl.BlockSpec((1, 4 * H), lambda i: (0, 0)),
        ],
        out_specs=[pl.BlockSpec((_RB, H), lambda i: (i, 0))] + _qkvs_specs(),
        out_shape=(jax.ShapeDtypeStruct((N, H), _f32),) + _qkvs_out(),
    )(x, we, be, wcat, bcat)


def _tc_step(y, aggA, aggB, sp, dtv, wcat, bcat):
    return pl.pallas_call(
        _step_body,
        grid=(N // _RB,),
        in_specs=[
            pl.BlockSpec((_RB, H), lambda i: (i, 0)),
            pl.BlockSpec((_RB, 128), lambda i: (i, 0)),
            pl.BlockSpec((_RB, 128), lambda i: (i, 0)),
            pl.BlockSpec((_RB, H), lambda i: (i, 0)),
            pl.BlockSpec((1, 1), lambda i: (0, 0)),
            pl.BlockSpec((H, 4 * H), lambda i: (0, 0)),
            pl.BlockSpec((1, 4 * H), lambda i: (0, 0)),
        ],
        out_specs=[pl.BlockSpec((_RB, H), lambda i: (i, 0))] + _qkvs_specs(),
        out_shape=(jax.ShapeDtypeStruct((N, H), _f32),) + _qkvs_out(),
    )(y, aggA, aggB, sp, dtv, wcat, bcat)


def _tc_final(y, aggA, aggB, sp, dtv):
    return pl.pallas_call(
        _final_body,
        grid=(N // _RB,),
        in_specs=[
            pl.BlockSpec((_RB, H), lambda i: (i, 0)),
            pl.BlockSpec((_RB, 128), lambda i: (i, 0)),
            pl.BlockSpec((_RB, 128), lambda i: (i, 0)),
            pl.BlockSpec((_RB, H), lambda i: (i, 0)),
            pl.BlockSpec((1, 1), lambda i: (0, 0)),
        ],
        out_specs=pl.BlockSpec((_RB, H), lambda i: (i, 0)),
        out_shape=jax.ShapeDtypeStruct((N, H), _f32),
    )(y, aggA, aggB, sp, dtv)


def kernel(x, edge_index, W_emb, b_emb, Wq, bq, Wk, bk, Wv, bv, Ws, bs):
    src = edge_index[0]
    dst = edge_index[1]
    wcat = jnp.concatenate([Wq[1:], Wk[1:], Wv[1:], Ws[1:]], axis=1)
    w0 = jnp.concatenate([Wq[0], Wk[0], Wv[0], Ws[0]])
    bcat = jnp.concatenate([bq, bk, bv, bs])
    ts = jnp.linspace(0.0, 1.0, N_STEPS)

    b0 = (bcat + ts[0] * w0)[None, :]
    h, q, k, vh, sp = _tc_first(x, W_emb, b_emb[None, :], wcat, b0)
    ys = [h]
    y = h
    for i in range(N_STEPS - 1):
        e, denp = _sc_scores(q, k, src, dst)
        den2 = _tc_densum(denp)
        agg = _sc_agg(vh.reshape(2 * N, 128), src, dst, e, den2)
        dtv = (ts[i + 1] - ts[i]).reshape(1, 1)
        if i < N_STEPS - 2:
            bi = (bcat + ts[i + 1] * w0)[None, :]
            y, q, k, vh, sp = _tc_step(y, agg[:N], agg[N:], sp, dtv, wcat, bi)
        else:
            y = _tc_final(y, agg[:N], agg[N:], sp, dtv)
        ys.append(y)
    return jnp.stack(ys, axis=0)
